# baseline (speedup 1.0000x reference)
import functools

import jax
import jax.numpy as jnp
from jax import lax
from jax.experimental import pallas as pl
from jax.experimental.pallas import tpu as pltpu
from jax.experimental.pallas import tpu_sc as plsc

EMB_DIM = 64
BATCH = 16384
NEG = 20
NTILES = 32
ITEMS_PER_TILE = BATCH // NTILES
SUB = 16
NSUB = ITEMS_PER_TILE // SUB
NEG_ROWS = SUB * NEG
NEG_PARTS = 4
PART = NEG_ROWS // NEG_PARTS
OUT_COLS = 48
NCOLS = 2 + 2 * NEG


def _sc_scores(u_l, u_r, v_l, v_r, pu2, pvl2, pvr2, nl2, nr2):
    mesh = plsc.VectorSubcoreMesh(core_axis_name="c", subcore_axis_name="s")

    gather_bufs = []
    for _ in range(2):
        gather_bufs += [
            pltpu.VMEM((SUB, EMB_DIM), jnp.float32),
            pltpu.VMEM((SUB, EMB_DIM), jnp.float32),
            pltpu.VMEM((SUB, EMB_DIM), jnp.float32),
            pltpu.VMEM((SUB, EMB_DIM), jnp.float32),
            pltpu.VMEM((NEG_ROWS, EMB_DIM), jnp.float32),
            pltpu.VMEM((NEG_ROWS, EMB_DIM), jnp.float32),
        ]

    @functools.partial(
        pl.kernel,
        out_type=jax.ShapeDtypeStruct((BATCH // SUB, OUT_COLS, SUB), jnp.float32),
        mesh=mesh,
        compiler_params=pltpu.CompilerParams(
            needs_layout_passes=False, use_tc_tiling_on_sc=False),
        scratch_types=[
            pltpu.VMEM((NSUB, SUB), jnp.int32),
            pltpu.VMEM((NSUB, SUB), jnp.int32),
            pltpu.VMEM((NSUB, SUB), jnp.int32),
            pltpu.VMEM((NSUB * NEG_PARTS, PART), jnp.int32),
            pltpu.VMEM((NSUB * NEG_PARTS, PART), jnp.int32),
        ] + gather_bufs + [
            pltpu.VMEM((OUT_COLS, SUB), jnp.float32),
            pltpu.VMEM((OUT_COLS, SUB), jnp.float32),
            pltpu.SemaphoreType.DMA,
            pltpu.SemaphoreType.DMA,
            pltpu.SemaphoreType.DMA,
            pltpu.SemaphoreType.DMA,
        ],
    )
    def k(u_l_h, u_r_h, v_l_h, v_r_h, pu_h, pvl_h, pvr_h, nl_h, nr_h, out_h,
          pu_v, pvl_v, pvr_v, nl_v, nr_v,
          eulA, eurA, evlA, evrA, enlA, enrA,
          eulB, eurB, evlB, evrB, enlB, enrB,
          scoA, scoB, semA, semB, semOA, semOB):
        wid = lax.axis_index("s") * 2 + lax.axis_index("c")

        pltpu.sync_copy(pu_h.at[pl.ds(wid * NSUB, NSUB)], pu_v)
        pltpu.sync_copy(pvl_h.at[pl.ds(wid * NSUB, NSUB)], pvl_v)
        pltpu.sync_copy(pvr_h.at[pl.ds(wid * NSUB, NSUB)], pvr_v)
        nrows = NSUB * NEG_PARTS
        pltpu.sync_copy(nl_h.at[pl.ds(wid * nrows, nrows)], nl_v)
        pltpu.sync_copy(nr_h.at[pl.ds(wid * nrows, nrows)], nr_v)

        lane = lax.iota(jnp.int32, 16)
        lane20pn = [lane * NEG + n for n in range(NEG)]
        zeros = jnp.zeros((16,), jnp.float32)
        for sco in (scoA, scoB):
            for c in range(NCOLS, OUT_COLS):
                sco[c, :] = zeros

        def fire(j, bufs, sem):
            eul, eur, evl, evr, enl, enr = bufs
            pltpu.async_copy(u_l_h.at[pu_v.at[j]], eul, sem)
            pltpu.async_copy(u_r_h.at[pu_v.at[j]], eur, sem)
            pltpu.async_copy(v_l_h.at[pvl_v.at[j]], evl, sem)
            pltpu.async_copy(v_r_h.at[pvr_v.at[j]], evr, sem)
            for p in range(NEG_PARTS):
                pltpu.async_copy(v_l_h.at[nl_v.at[j * NEG_PARTS + p]],
                                 enl.at[pl.ds(p * PART, PART)], sem)
                pltpu.async_copy(v_r_h.at[nr_v.at[j * NEG_PARTS + p]],
                                 enr.at[pl.ds(p * PART, PART)], sem)

        def drain(bufs, sem):
            eul, eur, evl, evr, enl, enr = bufs
            pltpu.make_async_copy(u_l_h.at[pl.ds(0, SUB)], eul, sem).wait()
            pltpu.make_async_copy(u_r_h.at[pl.ds(0, SUB)], eur, sem).wait()
            pltpu.make_async_copy(v_l_h.at[pl.ds(0, SUB)], evl, sem).wait()
            pltpu.make_async_copy(v_r_h.at[pl.ds(0, SUB)], evr, sem).wait()
            for p in range(NEG_PARTS):
                pltpu.make_async_copy(v_l_h.at[pl.ds(0, PART)],
                                      enl.at[pl.ds(p * PART, PART)], sem).wait()
                pltpu.make_async_copy(v_r_h.at[pl.ds(0, PART)],
                                      enr.at[pl.ds(p * PART, PART)], sem).wait()

        bufsA = (eulA, eurA, evlA, evrA, enlA, enrA)
        bufsB = (eulB, eurB, evlB, evrB, enlB, enrB)

        fire(0, bufsA, semA)

        def compute_inner(j, bufs, sco, semO, t):
            eul, eur, evl, evr, enl, enr = bufs
            @pl.when(t > 0)
            def _():
                pltpu.make_async_copy(sco, out_h.at[0], semO).wait()
            for c in range(NCOLS):
                sco[c, :] = zeros

            def side(eu, ev, en, pcol, ncol0):
                def dbody(dd, carry):
                    dvec = jnp.full((16,), dd, jnp.int32)
                    u = plsc.load_gather(eu, [lane, dvec])
                    v = plsc.load_gather(ev, [lane, dvec])
                    plsc.addupdate(sco.at[pcol], u * v)
                    for n in range(NEG):
                        nn = plsc.load_gather(en, [lane20pn[n], dvec])
                        plsc.addupdate(sco.at[ncol0 + n], u * nn)
                    return carry
                lax.fori_loop(0, EMB_DIM, dbody, 0)

            side(eul, evl, enl, 0, 2)
            side(eur, evr, enr, 1, 2 + NEG)
            sco[0, :] = -sco[0, :]
            sco[1, :] = -sco[1, :]
            pltpu.async_copy(sco, out_h.at[wid * NSUB + j], semO)

        def body(t, carry):
            j0 = 2 * t
            fire(j0 + 1, bufsB, semB)
            drain(bufsA, semA)
            compute_inner(j0, bufsA, scoA, semOA, t)

            @pl.when(j0 + 2 < NSUB)
            def _():
                fire(j0 + 2, bufsA, semA)
            drain(bufsB, semB)
            compute_inner(j0 + 1, bufsB, scoB, semOB, t)
            return carry

        lax.fori_loop(0, NSUB // 2, body, 0)
        pltpu.make_async_copy(scoA, out_h.at[0], semOA).wait()
        pltpu.make_async_copy(scoB, out_h.at[0], semOB).wait()

    return k(u_l, u_r, v_l, v_r, pu2, pvl2, pvr2, nl2, nr2)


def _tc_reduce(scores):
    def red(x_ref, o_ref):
        x = x_ref[...]
        s = jnp.clip(x, -10.0, 10.0)
        v = jnp.maximum(s, 0.0) + jnp.log(1.0 + jnp.exp(-jnp.abs(s)))
        col = lax.broadcasted_iota(jnp.int32, x.shape, 1)
        v = jnp.where(col < NCOLS * SUB, v, 0.0)
        o_ref[0, 0] = jnp.sum(v) * (1.0 / BATCH)

    out = pl.pallas_call(
        red,
        out_shape=jax.ShapeDtypeStruct((1, 1), jnp.float32),
        out_specs=pl.BlockSpec(memory_space=pltpu.SMEM),
    )(scores)
    return out[0, 0]


def kernel(pos_u, pos_v_l, pos_v_r, neg_v_l, neg_v_r,
           u_l_weight, u_r_weight, v_l_weight, v_r_weight):
    pu = pos_u.astype(jnp.int32).reshape(BATCH // SUB, SUB)
    pvl = pos_v_l.astype(jnp.int32).reshape(BATCH // SUB, SUB)
    pvr = pos_v_r.astype(jnp.int32).reshape(BATCH // SUB, SUB)
    nl = neg_v_l.astype(jnp.int32).reshape(-1, PART)
    nr = neg_v_r.astype(jnp.int32).reshape(-1, PART)
    scores = _sc_scores(u_l_weight, u_r_weight, v_l_weight, v_r_weight,
                        pu, pvl, pvr, nl, nr)
    return _tc_reduce(scores.reshape(BATCH // SUB, OUT_COLS * SUB))


# speedup vs baseline: 1.0850x; 1.0850x over previous
import functools

import jax
import jax.numpy as jnp
from jax import lax
from jax.experimental import pallas as pl
from jax.experimental.pallas import tpu as pltpu
from jax.experimental.pallas import tpu_sc as plsc

EMB_DIM = 64
BATCH = 16384
NEG = 20
NTILES = 32
ITEMS_PER_TILE = BATCH // NTILES
SUB = 16
NSUB = ITEMS_PER_TILE // SUB
NEG_ROWS = SUB * NEG
NEG_PARTS = 4
PART = NEG_ROWS // NEG_PARTS
OUT_COLS = 48
NCOLS = 2 + 2 * NEG


def _sc_scores(u_l, u_r, v_l, v_r, pu2, pvl2, pvr2, nl2, nr2):
    mesh = plsc.VectorSubcoreMesh(core_axis_name="c", subcore_axis_name="s")

    gather_bufs = []
    for _ in range(2):
        gather_bufs += [
            pltpu.VMEM((SUB, EMB_DIM), jnp.float32),
            pltpu.VMEM((SUB, EMB_DIM), jnp.float32),
            pltpu.VMEM((SUB, EMB_DIM), jnp.float32),
            pltpu.VMEM((SUB, EMB_DIM), jnp.float32),
            pltpu.VMEM((NEG_ROWS, EMB_DIM), jnp.float32),
            pltpu.VMEM((NEG_ROWS, EMB_DIM), jnp.float32),
        ]

    @functools.partial(
        pl.kernel,
        out_type=jax.ShapeDtypeStruct((BATCH // SUB, OUT_COLS, SUB), jnp.float32),
        mesh=mesh,
        compiler_params=pltpu.CompilerParams(
            needs_layout_passes=False, use_tc_tiling_on_sc=False),
        scratch_types=[
            pltpu.VMEM((NSUB, SUB), jnp.int32),
            pltpu.VMEM((NSUB, SUB), jnp.int32),
            pltpu.VMEM((NSUB, SUB), jnp.int32),
            pltpu.VMEM((NSUB * NEG_PARTS, PART), jnp.int32),
            pltpu.VMEM((NSUB * NEG_PARTS, PART), jnp.int32),
        ] + gather_bufs + [
            pltpu.VMEM((OUT_COLS, SUB), jnp.float32),
            pltpu.VMEM((OUT_COLS, SUB), jnp.float32),
            pltpu.SemaphoreType.DMA,
            pltpu.SemaphoreType.DMA,
            pltpu.SemaphoreType.DMA,
            pltpu.SemaphoreType.DMA,
        ],
    )
    def k(u_l_h, u_r_h, v_l_h, v_r_h, pu_h, pvl_h, pvr_h, nl_h, nr_h, out_h,
          pu_v, pvl_v, pvr_v, nl_v, nr_v,
          eulA, eurA, evlA, evrA, enlA, enrA,
          eulB, eurB, evlB, evrB, enlB, enrB,
          scoA, scoB, semA, semB, semOA, semOB):
        wid = lax.axis_index("s") * 2 + lax.axis_index("c")

        pltpu.sync_copy(pu_h.at[pl.ds(wid * NSUB, NSUB)], pu_v)
        pltpu.sync_copy(pvl_h.at[pl.ds(wid * NSUB, NSUB)], pvl_v)
        pltpu.sync_copy(pvr_h.at[pl.ds(wid * NSUB, NSUB)], pvr_v)
        nrows = NSUB * NEG_PARTS
        pltpu.sync_copy(nl_h.at[pl.ds(wid * nrows, nrows)], nl_v)
        pltpu.sync_copy(nr_h.at[pl.ds(wid * nrows, nrows)], nr_v)

        lane = lax.iota(jnp.int32, 16)
        zeros = jnp.zeros((16,), jnp.float32)
        for sco in (scoA, scoB):
            for c in range(NCOLS, OUT_COLS):
                sco[c, :] = zeros

        def fire(j, bufs, sem):
            eul, eur, evl, evr, enl, enr = bufs
            pltpu.async_copy(u_l_h.at[pu_v.at[j]], eul, sem)
            pltpu.async_copy(u_r_h.at[pu_v.at[j]], eur, sem)
            pltpu.async_copy(v_l_h.at[pvl_v.at[j]], evl, sem)
            pltpu.async_copy(v_r_h.at[pvr_v.at[j]], evr, sem)
            for p in range(NEG_PARTS):
                pltpu.async_copy(v_l_h.at[nl_v.at[j * NEG_PARTS + p]],
                                 enl.at[pl.ds(p * PART, PART)], sem)
                pltpu.async_copy(v_r_h.at[nr_v.at[j * NEG_PARTS + p]],
                                 enr.at[pl.ds(p * PART, PART)], sem)

        def drain(bufs, sem):
            eul, eur, evl, evr, enl, enr = bufs
            pltpu.make_async_copy(u_l_h.at[pl.ds(0, SUB)], eul, sem).wait()
            pltpu.make_async_copy(u_r_h.at[pl.ds(0, SUB)], eur, sem).wait()
            pltpu.make_async_copy(v_l_h.at[pl.ds(0, SUB)], evl, sem).wait()
            pltpu.make_async_copy(v_r_h.at[pl.ds(0, SUB)], evr, sem).wait()
            for p in range(NEG_PARTS):
                pltpu.make_async_copy(v_l_h.at[pl.ds(0, PART)],
                                      enl.at[pl.ds(p * PART, PART)], sem).wait()
                pltpu.make_async_copy(v_r_h.at[pl.ds(0, PART)],
                                      enr.at[pl.ds(p * PART, PART)], sem).wait()

        bufsA = (eulA, eurA, evlA, evrA, enlA, enrA)
        bufsB = (eulB, eurB, evlB, evrB, enlB, enrB)

        fire(0, bufsA, semA)

        DCHUNK = 16

        def compute_inner(j, bufs, sco, semO, t):
            eul, eur, evl, evr, enl, enr = bufs
            @pl.when(t > 0)
            def _():
                pltpu.make_async_copy(sco, out_h.at[0], semO).wait()
            for c in range(NCOLS):
                sco[c, :] = zeros

            def side(eu, ev, en, pcol, ncol0):
                def cbody(ch, carry):
                    base = ch * DCHUNK
                    pos = sco[pcol, :]
                    negs = [sco[ncol0 + n, :] for n in range(NEG)]
                    for di in range(DCHUNK):
                        dvec = jnp.full((16,), base + di, jnp.int32)
                        u = plsc.load_gather(eu, [lane, dvec])
                        v = plsc.load_gather(ev, [lane, dvec])
                        pos = pos + u * v
                        for n in range(NEG):
                            nn = plsc.load_gather(
                                en.at[pl.ds(n * SUB, SUB)], [lane, dvec])
                            negs[n] = negs[n] + u * nn
                    sco[pcol, :] = pos
                    for n in range(NEG):
                        sco[ncol0 + n, :] = negs[n]
                    return carry
                lax.fori_loop(0, EMB_DIM // DCHUNK, cbody, 0)

            side(eul, evl, enl, 0, 2)
            side(eur, evr, enr, 1, 2 + NEG)
            sco[0, :] = -sco[0, :]
            sco[1, :] = -sco[1, :]
            pltpu.async_copy(sco, out_h.at[wid * NSUB + j], semO)

        def body(t, carry):
            j0 = 2 * t
            fire(j0 + 1, bufsB, semB)
            drain(bufsA, semA)
            compute_inner(j0, bufsA, scoA, semOA, t)

            @pl.when(j0 + 2 < NSUB)
            def _():
                fire(j0 + 2, bufsA, semA)
            drain(bufsB, semB)
            compute_inner(j0 + 1, bufsB, scoB, semOB, t)
            return carry

        lax.fori_loop(0, NSUB // 2, body, 0)
        pltpu.make_async_copy(scoA, out_h.at[0], semOA).wait()
        pltpu.make_async_copy(scoB, out_h.at[0], semOB).wait()

    return k(u_l, u_r, v_l, v_r, pu2, pvl2, pvr2, nl2, nr2)


def _tc_reduce(scores):
    def red(x_ref, o_ref):
        x = x_ref[...]
        s = jnp.clip(x, -10.0, 10.0)
        v = jnp.maximum(s, 0.0) + jnp.log(1.0 + jnp.exp(-jnp.abs(s)))
        col = lax.broadcasted_iota(jnp.int32, x.shape, 1)
        v = jnp.where(col < NCOLS * SUB, v, 0.0)
        o_ref[0, 0] = jnp.sum(v) * (1.0 / BATCH)

    out = pl.pallas_call(
        red,
        out_shape=jax.ShapeDtypeStruct((1, 1), jnp.float32),
        out_specs=pl.BlockSpec(memory_space=pltpu.SMEM),
    )(scores)
    return out[0, 0]


def kernel(pos_u, pos_v_l, pos_v_r, neg_v_l, neg_v_r,
           u_l_weight, u_r_weight, v_l_weight, v_r_weight):
    pu = pos_u.astype(jnp.int32).reshape(BATCH // SUB, SUB)
    pvl = pos_v_l.astype(jnp.int32).reshape(BATCH // SUB, SUB)
    pvr = pos_v_r.astype(jnp.int32).reshape(BATCH // SUB, SUB)
    nl = (neg_v_l.astype(jnp.int32).reshape(BATCH // SUB, SUB, NEG)
          .transpose(0, 2, 1).reshape(-1, PART))
    nr = (neg_v_r.astype(jnp.int32).reshape(BATCH // SUB, SUB, NEG)
          .transpose(0, 2, 1).reshape(-1, PART))
    scores = _sc_scores(u_l_weight, u_r_weight, v_l_weight, v_r_weight,
                        pu, pvl, pvr, nl, nr)
    return _tc_reduce(scores.reshape(BATCH // SUB, OUT_COLS * SUB))


# speedup vs baseline: 1.1245x; 1.0364x over previous
import functools

import jax
import jax.numpy as jnp
from jax import lax
from jax.experimental import pallas as pl
from jax.experimental.pallas import tpu as pltpu
from jax.experimental.pallas import tpu_sc as plsc

EMB_DIM = 64
BATCH = 16384
NEG = 20
NTILES = 32
ITEMS_PER_TILE = BATCH // NTILES
SUB = 32
NSUB = ITEMS_PER_TILE // SUB
NEG_ROWS = SUB * NEG
PART = 128
NEG_PARTS = NEG_ROWS // PART
OUT_COLS = 48
NCOLS = 2 + 2 * NEG
LG = 16
DCHUNK = 16


def _sc_scores(u_l, u_r, v_l, v_r, pu2, pvl2, pvr2, nl2, nr2):
    mesh = plsc.VectorSubcoreMesh(core_axis_name="c", subcore_axis_name="s")

    @functools.partial(
        pl.kernel,
        out_type=jax.ShapeDtypeStruct((BATCH // SUB, OUT_COLS, SUB), jnp.float32),
        mesh=mesh,
        compiler_params=pltpu.CompilerParams(
            needs_layout_passes=False, use_tc_tiling_on_sc=False),
        scratch_types=[
            pltpu.VMEM((NSUB, SUB), jnp.int32),
            pltpu.VMEM((NSUB, SUB), jnp.int32),
            pltpu.VMEM((NSUB, SUB), jnp.int32),
            pltpu.VMEM((NSUB * NEG_PARTS, PART), jnp.int32),
            pltpu.VMEM((NSUB * NEG_PARTS, PART), jnp.int32),
            pltpu.VMEM((SUB, EMB_DIM), jnp.float32),
            pltpu.VMEM((SUB, EMB_DIM), jnp.float32),
            pltpu.VMEM((NEG_ROWS, EMB_DIM), jnp.float32),
            pltpu.VMEM((SUB, EMB_DIM), jnp.float32),
            pltpu.VMEM((SUB, EMB_DIM), jnp.float32),
            pltpu.VMEM((NEG_ROWS, EMB_DIM), jnp.float32),
            pltpu.VMEM((OUT_COLS, SUB), jnp.float32),
            pltpu.VMEM((OUT_COLS, SUB), jnp.float32),
            pltpu.SemaphoreType.DMA,
            pltpu.SemaphoreType.DMA,
            pltpu.SemaphoreType.DMA,
            pltpu.SemaphoreType.DMA,
        ],
    )
    def k(u_l_h, u_r_h, v_l_h, v_r_h, pu_h, pvl_h, pvr_h, nl_h, nr_h, out_h,
          pu_v, pvl_v, pvr_v, nl_v, nr_v,
          eul, evl, enl, eur, evr, enr,
          scoA, scoB, semL, semR, semOA, semOB):
        wid = lax.axis_index("s") * 2 + lax.axis_index("c")

        pltpu.sync_copy(pu_h.at[pl.ds(wid * NSUB, NSUB)], pu_v)
        pltpu.sync_copy(pvl_h.at[pl.ds(wid * NSUB, NSUB)], pvl_v)
        pltpu.sync_copy(pvr_h.at[pl.ds(wid * NSUB, NSUB)], pvr_v)
        nrows = NSUB * NEG_PARTS
        pltpu.sync_copy(nl_h.at[pl.ds(wid * nrows, nrows)], nl_v)
        pltpu.sync_copy(nr_h.at[pl.ds(wid * nrows, nrows)], nr_v)

        lane = lax.iota(jnp.int32, 16)
        zeros = jnp.zeros((16,), jnp.float32)
        for sco in (scoA, scoB):
            for c in range(NCOLS, OUT_COLS):
                sco[c, pl.ds(0, LG)] = zeros
                sco[c, pl.ds(LG, LG)] = zeros

        def fire(j, u_h, v_h, p_v, n_v, eu, ev, en, sem):
            pltpu.async_copy(u_h.at[pu_v.at[j]], eu, sem)
            pltpu.async_copy(v_h.at[p_v.at[j]], ev, sem)
            for p in range(NEG_PARTS):
                pltpu.async_copy(v_h.at[n_v.at[j * NEG_PARTS + p]],
                                 en.at[pl.ds(p * PART, PART)], sem)

        def drain(u_h, v_h, eu, ev, en, sem):
            pltpu.make_async_copy(u_h.at[pl.ds(0, SUB)], eu, sem).wait()
            pltpu.make_async_copy(v_h.at[pl.ds(0, SUB)], ev, sem).wait()
            for p in range(NEG_PARTS):
                pltpu.make_async_copy(v_h.at[pl.ds(0, PART)],
                                      en.at[pl.ds(p * PART, PART)], sem).wait()

        def fire_l(j):
            fire(j, u_l_h, v_l_h, pvl_v, nl_v, eul, evl, enl, semL)

        def fire_r(j):
            fire(j, u_r_h, v_r_h, pvr_v, nr_v, eur, evr, enr, semR)

        def compute_side(sco, eu, ev, en, pcol, ncol0):
            for h in range(SUB // LG):
                hsl = pl.ds(h * LG, LG)
                eu_h = eu.at[pl.ds(h * LG, LG)]
                ev_h = ev.at[pl.ds(h * LG, LG)]
                groups = ((True, range(0, NEG // 2)),
                          (False, range(NEG // 2, NEG)))

                def cbody(ch, carry):
                    base = ch * DCHUNK
                    for has_pos, nrange in groups:
                        accs = {}
                        if has_pos:
                            accs["p"] = sco[pcol, hsl]
                        for n in nrange:
                            accs[n] = sco[ncol0 + n, hsl]
                        for di in range(DCHUNK):
                            dvec = jnp.full((16,), base + di, jnp.int32)
                            u = plsc.load_gather(eu_h, [lane, dvec])
                            if has_pos:
                                v = plsc.load_gather(ev_h, [lane, dvec])
                                accs["p"] = accs["p"] + u * v
                            for n in nrange:
                                nn = plsc.load_gather(
                                    en.at[pl.ds(n * SUB + h * LG, LG)],
                                    [lane, dvec])
                                accs[n] = accs[n] + u * nn
                        if has_pos:
                            sco[pcol, hsl] = accs["p"]
                        for n in nrange:
                            sco[ncol0 + n, hsl] = accs[n]
                    return carry

                lax.fori_loop(0, EMB_DIM // DCHUNK, cbody, 0)
                sco[pcol, hsl] = -sco[pcol, hsl]

        def zero_live(sco):
            for c in range(NCOLS):
                sco[c, pl.ds(0, LG)] = zeros
                sco[c, pl.ds(LG, LG)] = zeros

        def sub_chunk(j, sco, semO, t):
            fire_r(j)
            drain(u_l_h, v_l_h, eul, evl, enl, semL)

            @pl.when(t > 0)
            def _():
                pltpu.make_async_copy(sco, out_h.at[0], semO).wait()
            zero_live(sco)
            compute_side(sco, eul, evl, enl, 0, 2)

            @pl.when(j + 1 < NSUB)
            def _():
                fire_l(j + 1)
            drain(u_r_h, v_r_h, eur, evr, enr, semR)
            compute_side(sco, eur, evr, enr, 1, 2 + NEG)
            pltpu.async_copy(sco, out_h.at[wid * NSUB + j], semO)

        fire_l(0)

        def body(t, carry):
            sub_chunk(2 * t, scoA, semOA, t)
            sub_chunk(2 * t + 1, scoB, semOB, t)
            return carry

        lax.fori_loop(0, NSUB // 2, body, 0)
        pltpu.make_async_copy(scoA, out_h.at[0], semOA).wait()
        pltpu.make_async_copy(scoB, out_h.at[0], semOB).wait()

    return k(u_l, u_r, v_l, v_r, pu2, pvl2, pvr2, nl2, nr2)


def _tc_reduce(scores):
    def red(x_ref, o_ref):
        x = x_ref[...]
        s = jnp.clip(x, -10.0, 10.0)
        v = jnp.maximum(s, 0.0) + jnp.log(1.0 + jnp.exp(-jnp.abs(s)))
        col = lax.broadcasted_iota(jnp.int32, x.shape, 1)
        v = jnp.where(col < NCOLS * SUB, v, 0.0)
        o_ref[0, 0] = jnp.sum(v) * (1.0 / BATCH)

    out = pl.pallas_call(
        red,
        out_shape=jax.ShapeDtypeStruct((1, 1), jnp.float32),
        out_specs=pl.BlockSpec(memory_space=pltpu.SMEM),
    )(scores)
    return out[0, 0]


def kernel(pos_u, pos_v_l, pos_v_r, neg_v_l, neg_v_r,
           u_l_weight, u_r_weight, v_l_weight, v_r_weight):
    pu = pos_u.astype(jnp.int32).reshape(BATCH // SUB, SUB)
    pvl = pos_v_l.astype(jnp.int32).reshape(BATCH // SUB, SUB)
    pvr = pos_v_r.astype(jnp.int32).reshape(BATCH // SUB, SUB)
    nl = (neg_v_l.astype(jnp.int32).reshape(BATCH // SUB, SUB, NEG)
          .transpose(0, 2, 1).reshape(-1, PART))
    nr = (neg_v_r.astype(jnp.int32).reshape(BATCH // SUB, SUB, NEG)
          .transpose(0, 2, 1).reshape(-1, PART))
    scores = _sc_scores(u_l_weight, u_r_weight, v_l_weight, v_r_weight,
                        pu, pvl, pvr, nl, nr)
    return _tc_reduce(scores.reshape(BATCH // SUB, OUT_COLS * SUB))
